# baseline (speedup 1.0000x reference)
import functools

import jax
import jax.numpy as jnp
from jax import lax
from jax.experimental import pallas as pl
from jax.experimental.pallas import tpu as pltpu
from jax.experimental.pallas import tpu_sc as plsc

NC = 2
NS = 16
L = 16
NW = NC * NS

BATCH = 16384
D = 64
B_PER_W = BATCH // NW
CHUNK = 128
N_CHUNK = B_PER_W // CHUNK
GROUPS = B_PER_W // L


def _body(uid_hbm, bid_hbm, user_hbm, book_hbm, wu_hbm, wb_hbm, bias_hbm,
          out_hbm,
          idx_u, idx_b, rows_u, rows_b, wu_v, wb_v, bias_v, out_v, sem):
    c = lax.axis_index("c")
    s = lax.axis_index("s")
    wid = s * NC + c
    chunk0 = wid * N_CHUNK

    pltpu.sync_copy(uid_hbm.at[pl.ds(chunk0, N_CHUNK)], idx_u)
    pltpu.sync_copy(bid_hbm.at[pl.ds(chunk0, N_CHUNK)], idx_b)
    pltpu.sync_copy(wu_hbm, wu_v)
    pltpu.sync_copy(wb_hbm, wb_v)
    pltpu.sync_copy(bias_hbm, bias_v)

    descs = []
    for j in range(N_CHUNK):
        descs.append(pltpu.async_copy(
            user_hbm.at[idx_u.at[j]], rows_u.at[pl.ds(j * CHUNK, CHUNK)], sem))
        descs.append(pltpu.async_copy(
            book_hbm.at[idx_b.at[j]], rows_b.at[pl.ds(j * CHUNK, CHUNK)], sem))
    for d in descs:
        d.wait()

    iota16 = lax.iota(jnp.int32, 16)

    def group(g, _):
        rows16 = iota16 + g * L
        acc = bias_v[...]
        for d in range(D):
            col = jnp.full((L,), d, jnp.int32)
            vu = plsc.load_gather(rows_u, [rows16, col])
            vb = plsc.load_gather(rows_b, [rows16, col])
            acc = acc + vu * wu_v[d] + vb * wb_v[d]
        out_v[pl.ds(g * L, L)] = acc
        return _

    lax.fori_loop(0, GROUPS, group, None)

    pltpu.sync_copy(out_v, out_hbm.at[pl.ds(wid * B_PER_W, B_PER_W)])


@functools.cache
def _sc_kernel():
    return pl.kernel(
        _body,
        out_type=jax.ShapeDtypeStruct((BATCH,), jnp.float32),
        mesh=plsc.VectorSubcoreMesh(core_axis_name="c", subcore_axis_name="s",
                                    num_cores=NC, num_subcores=NS),
        scratch_types=[
            pltpu.VMEM((N_CHUNK, CHUNK), jnp.int32),
            pltpu.VMEM((N_CHUNK, CHUNK), jnp.int32),
            pltpu.VMEM((B_PER_W, D), jnp.float32),
            pltpu.VMEM((B_PER_W, D), jnp.float32),
            pltpu.VMEM((D, L), jnp.float32),
            pltpu.VMEM((D, L), jnp.float32),
            pltpu.VMEM((L,), jnp.float32),
            pltpu.VMEM((B_PER_W,), jnp.float32),
            pltpu.SemaphoreType.DMA,
        ],
        compiler_params=pltpu.CompilerParams(needs_layout_passes=False,
                                             use_tc_tiling_on_sc=False),
    )


@jax.jit
def kernel(user_ids, book_ids, user_table, book_table, fc_w, fc_b):
    uid2 = user_ids.astype(jnp.int32).reshape(BATCH // CHUNK, CHUNK)
    bid2 = book_ids.astype(jnp.int32).reshape(BATCH // CHUNK, CHUNK)
    wu = jnp.broadcast_to(fc_w[:D], (D, L)).astype(jnp.float32)
    wb = jnp.broadcast_to(fc_w[D:], (D, L)).astype(jnp.float32)
    bias = jnp.full((L,), fc_b[0], jnp.float32)
    return _sc_kernel()(uid2, bid2, user_table, book_table, wu, wb, bias)


# speedup vs baseline: 5.8017x; 5.8017x over previous
import functools

import jax
import jax.numpy as jnp
from jax import lax
from jax.experimental import pallas as pl
from jax.experimental.pallas import tpu as pltpu
from jax.experimental.pallas import tpu_sc as plsc

NC = 2
NS = 16
L = 16
NW = NC * NS

BATCH = 16384
D = 64
V = 1000000
B_PER_W = BATCH // NW
GROUPS = B_PER_W // L
CHUNK = 128
N_CHUNK = B_PER_W // CHUNK
ZROW = 16
BN = 8192
GRID = (V + BN - 1) // BN


def _tc_body(at_ref, bt_ref, wu_ref, wb_ref, zu_ref, zb_ref):
    zu_ref[...] = jnp.dot(wu_ref[...], at_ref[...],
                          preferred_element_type=jnp.float32)[0]
    zb_ref[...] = jnp.dot(wb_ref[...], bt_ref[...],
                          preferred_element_type=jnp.float32)[0]


@functools.cache
def _tc_matvec():
    return pl.pallas_call(
        _tc_body,
        grid=(GRID,),
        in_specs=[
            pl.BlockSpec((D, BN), lambda i: (0, i)),
            pl.BlockSpec((D, BN), lambda i: (0, i)),
            pl.BlockSpec((1, D), lambda i: (0, 0)),
            pl.BlockSpec((1, D), lambda i: (0, 0)),
        ],
        out_specs=[
            pl.BlockSpec((BN,), lambda i: (i,)),
            pl.BlockSpec((BN,), lambda i: (i,)),
        ],
        out_shape=[
            jax.ShapeDtypeStruct((V,), jnp.float32),
            jax.ShapeDtypeStruct((V,), jnp.float32),
        ],
    )


def _sc_body(zu_hbm, zb_hbm, uid_hbm, bid_hbm, bias_hbm,
             out_hbm,
             idx_u, idx_b, row_u, row_b, lane_u, lane_b,
             rows_u, rows_b, bias_v, out_v, sem):
    c = lax.axis_index("c")
    s = lax.axis_index("s")
    wid = s * NC + c
    base = wid * B_PER_W

    pltpu.sync_copy(uid_hbm.at[pl.ds(base, B_PER_W)], idx_u)
    pltpu.sync_copy(bid_hbm.at[pl.ds(base, B_PER_W)], idx_b)
    pltpu.sync_copy(bias_hbm, bias_v)

    for k in range(GROUPS):
        j, off = k // 8, (k % 8) * L
        vu = idx_u[pl.ds(k * L, L)]
        vb = idx_b[pl.ds(k * L, L)]
        row_u[j, pl.ds(off, L)] = vu >> 4
        row_b[j, pl.ds(off, L)] = vb >> 4
        lane_u[pl.ds(k * L, L)] = vu & 15
        lane_b[pl.ds(k * L, L)] = vb & 15

    descs = []
    for j in range(N_CHUNK):
        descs.append(pltpu.async_copy(
            zu_hbm.at[row_u.at[j]], rows_u.at[pl.ds(j * CHUNK, CHUNK)], sem))
        descs.append(pltpu.async_copy(
            zb_hbm.at[row_b.at[j]], rows_b.at[pl.ds(j * CHUNK, CHUNK)], sem))
    for d in descs:
        d.wait()

    iota16 = lax.iota(jnp.int32, L)

    def group(g, _):
        rids = iota16 + g * L
        vu = plsc.load_gather(rows_u, [rids, lane_u[pl.ds(g * L, L)]])
        vb = plsc.load_gather(rows_b, [rids, lane_b[pl.ds(g * L, L)]])
        out_v[pl.ds(g * L, L)] = vu + vb + bias_v[...]
        return _

    lax.fori_loop(0, GROUPS, group, None)

    pltpu.sync_copy(out_v, out_hbm.at[pl.ds(base, B_PER_W)])


@functools.cache
def _sc_gather():
    return pl.kernel(
        _sc_body,
        out_type=jax.ShapeDtypeStruct((BATCH,), jnp.float32),
        mesh=plsc.VectorSubcoreMesh(core_axis_name="c", subcore_axis_name="s",
                                    num_cores=NC, num_subcores=NS),
        scratch_types=[
            pltpu.VMEM((B_PER_W,), jnp.int32),
            pltpu.VMEM((B_PER_W,), jnp.int32),
            pltpu.VMEM((N_CHUNK, CHUNK), jnp.int32),
            pltpu.VMEM((N_CHUNK, CHUNK), jnp.int32),
            pltpu.VMEM((B_PER_W,), jnp.int32),
            pltpu.VMEM((B_PER_W,), jnp.int32),
            pltpu.VMEM((B_PER_W, ZROW), jnp.float32),
            pltpu.VMEM((B_PER_W, ZROW), jnp.float32),
            pltpu.VMEM((L,), jnp.float32),
            pltpu.VMEM((B_PER_W,), jnp.float32),
            pltpu.SemaphoreType.DMA,
        ],
        compiler_params=pltpu.CompilerParams(needs_layout_passes=False,
                                             use_tc_tiling_on_sc=False),
    )


@jax.jit
def kernel(user_ids, book_ids, user_table, book_table, fc_w, fc_b):
    uid = user_ids.astype(jnp.int32)
    bid = book_ids.astype(jnp.int32)
    wu = fc_w[:D, 0].reshape(1, D).astype(jnp.float32)
    wb = fc_w[D:, 0].reshape(1, D).astype(jnp.float32)
    zu, zb = _tc_matvec()(user_table.T, book_table.T, wu, wb)
    zu2 = zu.reshape(V // ZROW, ZROW)
    zb2 = zb.reshape(V // ZROW, ZROW)
    bias = jnp.full((L,), fc_b[0], jnp.float32)
    return _sc_gather()(zu2, zb2, uid, bid, bias)


# speedup vs baseline: 6.4654x; 1.1144x over previous
import functools

import jax
import jax.numpy as jnp
from jax import lax
from jax.experimental import pallas as pl
from jax.experimental.pallas import tpu as pltpu
from jax.experimental.pallas import tpu_sc as plsc

NC = 2
NS = 16
L = 16
NW = NC * NS

BATCH = 16384
D = 64
V = 1000000
B_PER_W = BATCH // NW
GROUPS = B_PER_W // L
CHUNK = 128
N_CHUNK = B_PER_W // CHUNK
ZROW = 16
BN = 32768
GRID = (V + BN - 1) // BN


def _tc_body(at_ref, bt_ref, wu_ref, wb_ref, zu_ref, zb_ref):
    zu_ref[...] = jnp.dot(wu_ref[...], at_ref[...],
                          preferred_element_type=jnp.float32)[0]
    zb_ref[...] = jnp.dot(wb_ref[...], bt_ref[...],
                          preferred_element_type=jnp.float32)[0]


@functools.cache
def _tc_matvec():
    return pl.pallas_call(
        _tc_body,
        grid=(GRID,),
        in_specs=[
            pl.BlockSpec((D, BN), lambda i: (0, i)),
            pl.BlockSpec((D, BN), lambda i: (0, i)),
            pl.BlockSpec((1, D), lambda i: (0, 0)),
            pl.BlockSpec((1, D), lambda i: (0, 0)),
        ],
        out_specs=[
            pl.BlockSpec((BN,), lambda i: (i,)),
            pl.BlockSpec((BN,), lambda i: (i,)),
        ],
        out_shape=[
            jax.ShapeDtypeStruct((V,), jnp.float32),
            jax.ShapeDtypeStruct((V,), jnp.float32),
        ],
    )


def _sc_body(zu_hbm, zb_hbm, uid_hbm, bid_hbm, bias_hbm,
             out_hbm,
             idx_u, idx_b, row_u, row_b, lane_u, lane_b,
             rows_u, rows_b, bias_v, out_v, sem):
    c = lax.axis_index("c")
    s = lax.axis_index("s")
    wid = s * NC + c
    base = wid * B_PER_W

    pltpu.sync_copy(uid_hbm.at[pl.ds(base, B_PER_W)], idx_u)
    pltpu.sync_copy(bid_hbm.at[pl.ds(base, B_PER_W)], idx_b)
    pltpu.sync_copy(bias_hbm, bias_v)

    for k in range(GROUPS):
        j, off = k // 8, (k % 8) * L
        vu = idx_u[pl.ds(k * L, L)]
        vb = idx_b[pl.ds(k * L, L)]
        row_u[j, pl.ds(off, L)] = vu >> 4
        row_b[j, pl.ds(off, L)] = vb >> 4
        lane_u[pl.ds(k * L, L)] = vu & 15
        lane_b[pl.ds(k * L, L)] = vb & 15

    descs = []
    for j in range(N_CHUNK):
        descs.append(pltpu.async_copy(
            zu_hbm.at[row_u.at[j]], rows_u.at[pl.ds(j * CHUNK, CHUNK)], sem))
        descs.append(pltpu.async_copy(
            zb_hbm.at[row_b.at[j]], rows_b.at[pl.ds(j * CHUNK, CHUNK)], sem))
    for d in descs:
        d.wait()

    iota16 = lax.iota(jnp.int32, L)

    def group(g, _):
        rids = iota16 + g * L
        vu = plsc.load_gather(rows_u, [rids, lane_u[pl.ds(g * L, L)]])
        vb = plsc.load_gather(rows_b, [rids, lane_b[pl.ds(g * L, L)]])
        out_v[pl.ds(g * L, L)] = vu + vb + bias_v[...]
        return _

    lax.fori_loop(0, GROUPS, group, None)

    pltpu.sync_copy(out_v, out_hbm.at[pl.ds(base, B_PER_W)])


@functools.cache
def _sc_gather():
    return pl.kernel(
        _sc_body,
        out_type=jax.ShapeDtypeStruct((BATCH,), jnp.float32),
        mesh=plsc.VectorSubcoreMesh(core_axis_name="c", subcore_axis_name="s",
                                    num_cores=NC, num_subcores=NS),
        scratch_types=[
            pltpu.VMEM((B_PER_W,), jnp.int32),
            pltpu.VMEM((B_PER_W,), jnp.int32),
            pltpu.VMEM((N_CHUNK, CHUNK), jnp.int32),
            pltpu.VMEM((N_CHUNK, CHUNK), jnp.int32),
            pltpu.VMEM((B_PER_W,), jnp.int32),
            pltpu.VMEM((B_PER_W,), jnp.int32),
            pltpu.VMEM((B_PER_W, ZROW), jnp.float32),
            pltpu.VMEM((B_PER_W, ZROW), jnp.float32),
            pltpu.VMEM((L,), jnp.float32),
            pltpu.VMEM((B_PER_W,), jnp.float32),
            pltpu.SemaphoreType.DMA,
        ],
        compiler_params=pltpu.CompilerParams(needs_layout_passes=False,
                                             use_tc_tiling_on_sc=False),
    )


@jax.jit
def kernel(user_ids, book_ids, user_table, book_table, fc_w, fc_b):
    uid = user_ids.astype(jnp.int32)
    bid = book_ids.astype(jnp.int32)
    wu = fc_w[:D, 0].reshape(1, D).astype(jnp.float32)
    wb = fc_w[D:, 0].reshape(1, D).astype(jnp.float32)
    zu, zb = _tc_matvec()(user_table.T, book_table.T, wu, wb)
    zu2 = zu.reshape(V // ZROW, ZROW)
    zb2 = zb.reshape(V // ZROW, ZROW)
    bias = jnp.full((L,), fc_b[0], jnp.float32)
    return _sc_gather()(zu2, zb2, uid, bid, bias)
